# Initial kernel scaffold; baseline (speedup 1.0000x reference)
import jax
import jax.numpy as jnp
from jax.experimental import pallas as pl


def kernel(x, weight, bias, sample_map, oh, ow):
    raise NotImplementedError("write your pallas kernel here")


# speedup vs baseline: 1.1821x; 1.1821x over previous
import functools

import jax
import jax.numpy as jnp
from jax import lax
from jax.experimental import pallas as pl
from jax.experimental.pallas import tpu as pltpu
from jax.experimental.pallas import tpu_sc as plsc

CIN = 96
COUT = 96
KT = 4
H = W = 384
P = H * W
N = KT * P
OH, OW = 2 * H, 2 * W
NPIX = OH * OW
CP = 128

RPX = 8 * W
GI = P // RPX
FROWS = N // 128

NTILES = 16
CHUNK = N // NTILES
CROWS = CHUNK // 128
SLAB = 13312
NSLAB = (NPIX + SLAB - 1) // SLAB
TAIL = NPIX - (NSLAB - 1) * SLAB
ACC_ROWS = SLAB + 16
STRIPE = SLAB // NTILES
TAIL_STRIPE = TAIL // NTILES
GB = 64
SLABS_PER_CORE = (NSLAB + 1) // 2
WSZ = 2048
WROWS = WSZ // 128
NWIN = CHUNK // WSZ
MBUF = WSZ + 2 * GB


def _tc_body(x_ref, w_ref, sm_ref, contrib_ref, flat_ref):
    wb = w_ref[...]
    for r in range(8):
        xr = x_ref[0, :, r, :]
        res = lax.dot_general(xr, wb, (((0,), (0,)), ((), ())),
                              preferred_element_type=jnp.float32)
        contrib_ref[pl.ds(r * W, W), :] = res
    sm = sm_ref[0]
    fl = sm[:, 1] * OW + sm[:, 0]
    flat_ref[...] = fl.reshape(RPX // 128, 128)


def _tc_matmul(x, w2p, smk):
    return pl.pallas_call(
        _tc_body,
        grid=(GI, KT),
        in_specs=[
            pl.BlockSpec((1, CIN, 8, W), lambda i, k: (0, 0, i, 0)),
            pl.BlockSpec((CIN, CP), lambda i, k: (0, k)),
            pl.BlockSpec((1, RPX, 2), lambda i, k: (k, i, 0)),
        ],
        out_specs=[
            pl.BlockSpec((RPX, CP), lambda i, k: (k * GI + i, 0)),
            pl.BlockSpec((RPX // 128, 128), lambda i, k: (k * GI + i, 0)),
        ],
        out_shape=[
            jax.ShapeDtypeStruct((N, CP), jnp.float32),
            jax.ShapeDtypeStruct((FROWS, 128), jnp.int32),
        ],
    )(x, w2p, smk)


def _tr_body(in_ref, out_ref):
    for r in range(8):
        blk = in_ref[pl.ds(r * OW, OW), :]
        t = jnp.transpose(blk, (1, 0))
        out_ref[:, r, :] = t[0:COUT, :]


def _transpose_out(out_scat):
    return pl.pallas_call(
        _tr_body,
        grid=(OH // 8,),
        in_specs=[pl.BlockSpec((8 * OW, CP), lambda i: (i, 0))],
        out_specs=pl.BlockSpec((COUT, 8, OW), lambda i: (0, i, 0)),
        out_shape=jax.ShapeDtypeStruct((COUT, OH, OW), jnp.float32),
    )(out_scat)


_MESH = plsc.VectorSubcoreMesh(core_axis_name="c", subcore_axis_name="s")


@functools.partial(
    pl.kernel,
    out_type=jax.ShapeDtypeStruct((NPIX, CP), jnp.float32),
    mesh=_MESH,
    scratch_types=[
        pltpu.VMEM_SHARED((ACC_ROWS, CP), jnp.float32),
    ],
    compiler_params=pltpu.CompilerParams(needs_layout_passes=False),
)
def _sc_scatter(contrib_hbm, flat_hbm, bias_hbm, out_hbm, accum):
    pl.run_scoped(
        functools.partial(_sc_body, contrib_hbm, flat_hbm, bias_hbm,
                          out_hbm, accum),
        pltpu.VMEM((WROWS, 128), jnp.int32),
        pltpu.VMEM((MBUF,), jnp.int32),
        pltpu.VMEM((MBUF,), jnp.int32),
        pltpu.VMEM((GB,), jnp.int32),
        pltpu.VMEM((GB,), jnp.int32),
        pltpu.VMEM((GB, CP), jnp.float32),
        pltpu.VMEM((64, CP), jnp.float32),
        pltpu.SemaphoreType.DMA,
    )


def _sc_body(contrib_hbm, flat_hbm, bias_hbm, out_hbm, accum,
             flat_win, dst_buf, src_buf, dst_batch, src_batch,
             staging, bias_v, sem):
    c = lax.axis_index("c")
    s = lax.axis_index("s")
    iota16 = lax.iota(jnp.int32, 16)
    pltpu.sync_copy(bias_hbm, bias_v)

    def flush_one(j, carry):
        base = j * GB
        for u in range(GB // 16):
            dst_batch[pl.ds(u * 16, 16)] = dst_buf[pl.ds(base + u * 16, 16)]
            src_batch[pl.ds(u * 16, 16)] = src_buf[pl.ds(base + u * 16, 16)]
        pltpu.async_copy(contrib_hbm.at[src_batch], staging, sem).wait()
        pltpu.sync_copy(staging, accum.at[dst_batch], add=True)
        return carry

    def slab_body(t, carry):
        slab_id = 2 * t + c

        @pl.when(slab_id < NSLAB)
        def _():
            lo = slab_id * SLAB
            is_tail = slab_id == NSLAB - 1
            not_tail = slab_id != NSLAB - 1

            @pl.when(not_tail)
            def _():
                def pc(p, carry):
                    pltpu.sync_copy(
                        bias_v,
                        accum.at[pl.ds(s * STRIPE + p * 64, 64)])
                    return carry
                lax.fori_loop(0, STRIPE // 64, pc, 0)

            @pl.when(is_tail)
            def _():
                def pc(p, carry):
                    pltpu.sync_copy(
                        bias_v,
                        accum.at[pl.ds(s * TAIL_STRIPE + p * 64, 64)])
                    return carry
                lax.fori_loop(0, TAIL_STRIPE // 64, pc, 0)

            plsc.subcore_barrier()

            def win_body(w, cnt):
                pltpu.sync_copy(
                    flat_hbm.at[pl.ds(s * CROWS + w * WROWS, WROWS)],
                    flat_win)
                src_base = s * CHUNK + w * WSZ

                def scan_body(i, carry2):
                    cnt2, src_v = carry2
                    v = flat_win[i // 8, pl.ds((i % 8) * 16, 16)]
                    off = v - lo
                    b1 = lax.shift_right_logical(off, 31)
                    b2 = lax.shift_right_logical((SLAB - 1) - off, 31)
                    mi = (1 - b1) - b2
                    pref = plsc.cumsum(mi)
                    real_pos = cnt2 + pref - 1
                    pos = real_pos * mi + (WSZ + GB + iota16) * (1 - mi)
                    plsc.store_scatter(dst_buf, [pos], off)
                    plsc.store_scatter(src_buf, [pos], src_v)
                    return cnt2 + pref[15], src_v + 16

                cnt, _ = lax.fori_loop(0, WSZ // 16, scan_body,
                                       (cnt, src_base + iota16))

                nfl = cnt // GB
                lax.fori_loop(0, nfl, flush_one, 0)
                rem_base = nfl * GB
                for u in range(GB // 16):
                    dst_batch[pl.ds(u * 16, 16)] = dst_buf[pl.ds(rem_base + u * 16, 16)]
                    src_batch[pl.ds(u * 16, 16)] = src_buf[pl.ds(rem_base + u * 16, 16)]
                for u in range(GB // 16):
                    dst_buf[pl.ds(u * 16, 16)] = dst_batch[pl.ds(u * 16, 16)]
                    src_buf[pl.ds(u * 16, 16)] = src_batch[pl.ds(u * 16, 16)]
                return cnt - nfl * GB

            cnt = lax.fori_loop(0, NWIN, win_body, jnp.int32(0))

            pad_dst = jnp.full((16,), SLAB, jnp.int32)
            pad_src = jnp.zeros((16,), jnp.int32)
            for u in range(GB // 16):
                dst_buf[pl.ds(cnt + u * 16, 16)] = pad_dst
                src_buf[pl.ds(cnt + u * 16, 16)] = pad_src
            lax.fori_loop(0, (cnt + (GB - 1)) // GB, flush_one, 0)

            plsc.subcore_barrier()

            @pl.when(not_tail)
            def _():
                def pc(p, carry):
                    pltpu.sync_copy(
                        accum.at[pl.ds(s * STRIPE + p * 64, 64)],
                        out_hbm.at[pl.ds(lo + s * STRIPE + p * 64, 64)])
                    return carry
                lax.fori_loop(0, STRIPE // 64, pc, 0)

            @pl.when(is_tail)
            def _():
                def pc(p, carry):
                    pltpu.sync_copy(
                        accum.at[pl.ds(s * TAIL_STRIPE + p * 64, 64)],
                        out_hbm.at[pl.ds(lo + s * TAIL_STRIPE + p * 64, 64)])
                    return carry
                lax.fori_loop(0, TAIL_STRIPE // 64, pc, 0)

            plsc.subcore_barrier()

        return carry

    lax.fori_loop(0, SLABS_PER_CORE, slab_body, 0)


def kernel(x, weight, bias, sample_map, oh, ow):
    wt = jnp.transpose(weight, (0, 2, 1))
    w2p = jnp.pad(wt, ((0, 0), (0, 0), (0, CP - COUT))).reshape(CIN, KT * CP)
    smk = sample_map.reshape(P, KT, 2).transpose(1, 0, 2)

    contrib, flat2d = _tc_matmul(x, w2p, smk)
    biasp = jnp.pad(bias, (0, CP - COUT))
    bias_tile = jnp.tile(biasp[None, :], (64, 1))

    out_scat = _sc_scatter(contrib, flat2d, bias_tile)
    out = _transpose_out(out_scat)
    return out[None]


# speedup vs baseline: 1.2191x; 1.0314x over previous
import functools

import jax
import jax.numpy as jnp
from jax import lax
from jax.experimental import pallas as pl
from jax.experimental.pallas import tpu as pltpu
from jax.experimental.pallas import tpu_sc as plsc

CIN = 96
COUT = 96
KT = 4
H = W = 384
P = H * W
N = KT * P
OH, OW = 2 * H, 2 * W
NPIX = OH * OW
CP = 128

RPX = 8 * W
GI = P // RPX
FROWS = N // 128

NTILES = 16
CHUNK = N // NTILES
CROWS = CHUNK // 128
SLAB = 12288
NSLAB = (NPIX + SLAB - 1) // SLAB
TAIL = NPIX - (NSLAB - 1) * SLAB
ACC_ROWS = SLAB + 16
STRIPE = SLAB // NTILES
TAIL_STRIPE = TAIL // NTILES
GB = 64
SLABS_PER_CORE = (NSLAB + 1) // 2
WSZ = 2048
WROWS = WSZ // 128
NWIN = CHUNK // WSZ
MBUF = WSZ + 2 * GB


def _tc_body(x_ref, w_ref, sm_ref, contrib_ref, flat_ref):
    wb = w_ref[...]
    for r in range(8):
        xr = x_ref[0, :, r, :]
        res = lax.dot_general(xr, wb, (((0,), (0,)), ((), ())),
                              preferred_element_type=jnp.float32)
        contrib_ref[pl.ds(r * W, W), :] = res
    sm = sm_ref[0]
    fl = sm[:, 1] * OW + sm[:, 0]
    flat_ref[...] = fl.reshape(RPX // 128, 128)


def _tc_matmul(x, w2p, smk):
    return pl.pallas_call(
        _tc_body,
        grid=(GI, KT),
        in_specs=[
            pl.BlockSpec((1, CIN, 8, W), lambda i, k: (0, 0, i, 0)),
            pl.BlockSpec((CIN, CP), lambda i, k: (0, k)),
            pl.BlockSpec((1, RPX, 2), lambda i, k: (k, i, 0)),
        ],
        out_specs=[
            pl.BlockSpec((RPX, CP), lambda i, k: (k * GI + i, 0)),
            pl.BlockSpec((RPX // 128, 128), lambda i, k: (k * GI + i, 0)),
        ],
        out_shape=[
            jax.ShapeDtypeStruct((N, CP), jnp.float32),
            jax.ShapeDtypeStruct((FROWS, 128), jnp.int32),
        ],
    )(x, w2p, smk)


def _tr_body(in_ref, out_ref):
    for r in range(8):
        blk = in_ref[pl.ds(r * OW, OW), :]
        t = jnp.transpose(blk, (1, 0))
        out_ref[:, r, :] = t[0:COUT, :]


def _transpose_out(out_scat):
    return pl.pallas_call(
        _tr_body,
        grid=(OH // 8,),
        in_specs=[pl.BlockSpec((8 * OW, CP), lambda i: (i, 0))],
        out_specs=pl.BlockSpec((COUT, 8, OW), lambda i: (0, i, 0)),
        out_shape=jax.ShapeDtypeStruct((COUT, OH, OW), jnp.float32),
    )(out_scat)


_MESH = plsc.VectorSubcoreMesh(core_axis_name="c", subcore_axis_name="s")


@functools.partial(
    pl.kernel,
    out_type=jax.ShapeDtypeStruct((NPIX, CP), jnp.float32),
    mesh=_MESH,
    scratch_types=[
        pltpu.VMEM_SHARED((ACC_ROWS, CP), jnp.float32),
    ],
    compiler_params=pltpu.CompilerParams(needs_layout_passes=False),
)
def _sc_scatter(contrib_hbm, flat_hbm, bias_hbm, out_hbm, accum):
    pl.run_scoped(
        functools.partial(_sc_body, contrib_hbm, flat_hbm, bias_hbm,
                          out_hbm, accum),
        pltpu.VMEM((WROWS, 128), jnp.int32),
        pltpu.VMEM((MBUF,), jnp.int32),
        pltpu.VMEM((MBUF,), jnp.int32),
        pltpu.VMEM((2, GB), jnp.int32),
        pltpu.VMEM((2, GB), jnp.int32),
        pltpu.VMEM((2, GB, CP), jnp.float32),
        pltpu.VMEM((64, CP), jnp.float32),
        pltpu.SemaphoreType.DMA,
    )


def _sc_body(contrib_hbm, flat_hbm, bias_hbm, out_hbm, accum,
             flat_win, dst_buf, src_buf, dst_batch, src_batch,
             staging, bias_v, sem):
    c = lax.axis_index("c")
    s = lax.axis_index("s")
    iota16 = lax.iota(jnp.int32, 16)
    pltpu.sync_copy(bias_hbm, bias_v)

    def drain_prev(nfired):
        prev = lax.rem(nfired + 1, 2)

        @pl.when(nfired >= 1)
        def _():
            pltpu.make_async_copy(contrib_hbm.at[src_batch.at[prev]],
                                  staging.at[prev], sem).wait()
            pltpu.sync_copy(staging.at[prev], accum.at[dst_batch.at[prev]],
                            add=True)

    def flush_one(j, nfired):
        drain_prev(nfired)
        par = lax.rem(nfired, 2)
        base = j * GB
        for u in range(GB // 16):
            dst_batch[par, pl.ds(u * 16, 16)] = dst_buf[pl.ds(base + u * 16, 16)]
            src_batch[par, pl.ds(u * 16, 16)] = src_buf[pl.ds(base + u * 16, 16)]
        pltpu.async_copy(contrib_hbm.at[src_batch.at[par]], staging.at[par], sem)
        return nfired + 1

    def slab_body(t, carry):
        slab_id = 2 * t + c

        @pl.when(slab_id < NSLAB)
        def _():
            lo = slab_id * SLAB
            is_tail = slab_id == NSLAB - 1
            not_tail = slab_id != NSLAB - 1

            @pl.when(not_tail)
            def _():
                def pc(p, carry):
                    pltpu.sync_copy(
                        bias_v,
                        accum.at[pl.ds(s * STRIPE + p * 64, 64)])
                    return carry
                lax.fori_loop(0, STRIPE // 64, pc, 0)

            @pl.when(is_tail)
            def _():
                def pc(p, carry):
                    pltpu.sync_copy(
                        bias_v,
                        accum.at[pl.ds(s * TAIL_STRIPE + p * 64, 64)])
                    return carry
                lax.fori_loop(0, TAIL_STRIPE // 64, pc, 0)

            plsc.subcore_barrier()

            def win_body(w, carry2):
                cnt, nfired = carry2
                pltpu.sync_copy(
                    flat_hbm.at[pl.ds(s * CROWS + w * WROWS, WROWS)],
                    flat_win)
                src_base = s * CHUNK + w * WSZ

                def scan_body(i, carry3):
                    cnt2, src_v = carry3
                    v = flat_win[i // 8, pl.ds((i % 8) * 16, 16)]
                    off = v - lo
                    b1 = lax.shift_right_logical(off, 31)
                    b2 = lax.shift_right_logical((SLAB - 1) - off, 31)
                    mi = (1 - b1) - b2
                    pref = plsc.cumsum(mi)
                    real_pos = cnt2 + pref - 1
                    pos = real_pos * mi + (WSZ + GB + iota16) * (1 - mi)
                    plsc.store_scatter(dst_buf, [pos], off)
                    plsc.store_scatter(src_buf, [pos], src_v)
                    return cnt2 + pref[15], src_v + 16

                cnt, _ = lax.fori_loop(0, WSZ // 16, scan_body,
                                       (cnt, src_base + iota16))

                nfl = cnt // GB
                nfired = lax.fori_loop(0, nfl, flush_one, nfired)
                rem_base = nfl * GB
                for u in range(GB // 16):
                    flat_win[0, pl.ds(u * 16, 16)] = dst_buf[pl.ds(rem_base + u * 16, 16)]
                    flat_win[1, pl.ds(u * 16, 16)] = src_buf[pl.ds(rem_base + u * 16, 16)]
                for u in range(GB // 16):
                    dst_buf[pl.ds(u * 16, 16)] = flat_win[0, pl.ds(u * 16, 16)]
                    src_buf[pl.ds(u * 16, 16)] = flat_win[1, pl.ds(u * 16, 16)]
                return cnt - nfl * GB, nfired

            cnt, nfired = lax.fori_loop(0, NWIN, win_body,
                                        (jnp.int32(0), jnp.int32(0)))

            pad_dst = jnp.full((16,), SLAB, jnp.int32)
            pad_src = jnp.zeros((16,), jnp.int32)
            for u in range(GB // 16):
                dst_buf[pl.ds(cnt + u * 16, 16)] = pad_dst
                src_buf[pl.ds(cnt + u * 16, 16)] = pad_src
            nfired = lax.fori_loop(0, (cnt + (GB - 1)) // GB, flush_one, nfired)
            drain_prev(nfired)

            plsc.subcore_barrier()

            @pl.when(not_tail)
            def _():
                def pc(p, carry):
                    pltpu.sync_copy(
                        accum.at[pl.ds(s * STRIPE + p * 64, 64)],
                        out_hbm.at[pl.ds(lo + s * STRIPE + p * 64, 64)])
                    return carry
                lax.fori_loop(0, STRIPE // 64, pc, 0)

            @pl.when(is_tail)
            def _():
                def pc(p, carry):
                    pltpu.sync_copy(
                        accum.at[pl.ds(s * TAIL_STRIPE + p * 64, 64)],
                        out_hbm.at[pl.ds(lo + s * TAIL_STRIPE + p * 64, 64)])
                    return carry
                lax.fori_loop(0, TAIL_STRIPE // 64, pc, 0)

            plsc.subcore_barrier()

        return carry

    lax.fori_loop(0, SLABS_PER_CORE, slab_body, 0)


def kernel(x, weight, bias, sample_map, oh, ow):
    wt = jnp.transpose(weight, (0, 2, 1))
    w2p = jnp.pad(wt, ((0, 0), (0, 0), (0, CP - COUT))).reshape(CIN, KT * CP)
    smk = sample_map.reshape(P, KT, 2).transpose(1, 0, 2)

    contrib, flat2d = _tc_matmul(x, w2p, smk)
    biasp = jnp.pad(bias, (0, CP - COUT))
    bias_tile = jnp.tile(biasp[None, :], (64, 1))

    out_scat = _sc_scatter(contrib, flat2d, bias_tile)
    out = _transpose_out(out_scat)
    return out[None]


# speedup vs baseline: 1.3529x; 1.1097x over previous
import functools

import jax
import jax.numpy as jnp
from jax import lax
from jax.experimental import pallas as pl
from jax.experimental.pallas import tpu as pltpu
from jax.experimental.pallas import tpu_sc as plsc

CIN = 96
COUT = 96
KT = 4
H = W = 384
P = H * W
N = KT * P
OH, OW = 2 * H, 2 * W
NPIX = OH * OW
CP = 128

RPX = 8 * W
GI = P // RPX
FROWS = N // 128

NTILES = 16
CHUNK = N // NTILES
CROWS = CHUNK // 128
SLAB = 12288
NSLAB = (NPIX + SLAB - 1) // SLAB
TAIL = NPIX - (NSLAB - 1) * SLAB
ACC_ROWS = SLAB + 16
STRIPE = SLAB // NTILES
TAIL_STRIPE = TAIL // NTILES
GB = 64
SLABS_PER_CORE = (NSLAB + 1) // 2
WSZ = 2048
WROWS = WSZ // 128
NWIN = CHUNK // WSZ
MBUF = WSZ + 2 * GB


def _tc_body(x_ref, w_ref, sm_ref, contrib_ref, flat_ref):
    wb = w_ref[...]
    for r in range(8):
        xr = x_ref[0, :, r, :]
        res = lax.dot_general(xr, wb, (((0,), (0,)), ((), ())),
                              preferred_element_type=jnp.float32)
        contrib_ref[pl.ds(r * W, W), :] = res
    sm = sm_ref[0]
    fl = sm[:, 1] * OW + sm[:, 0]
    flat_ref[...] = fl.reshape(RPX // 128, 128)


def _tc_matmul(x, w2p, smk):
    return pl.pallas_call(
        _tc_body,
        grid=(GI, KT),
        in_specs=[
            pl.BlockSpec((1, CIN, 8, W), lambda i, k: (0, 0, i, 0)),
            pl.BlockSpec((CIN, CP), lambda i, k: (0, k)),
            pl.BlockSpec((1, RPX, 2), lambda i, k: (k, i, 0)),
        ],
        out_specs=[
            pl.BlockSpec((RPX, CP), lambda i, k: (k * GI + i, 0)),
            pl.BlockSpec((RPX // 128, 128), lambda i, k: (k * GI + i, 0)),
        ],
        out_shape=[
            jax.ShapeDtypeStruct((N, CP), jnp.float32),
            jax.ShapeDtypeStruct((FROWS, 128), jnp.int32),
        ],
    )(x, w2p, smk)


def _tr_body(in_ref, out_ref):
    for r in range(8):
        blk = in_ref[pl.ds(r * OW, OW), :]
        t = jnp.transpose(blk, (1, 0))
        out_ref[:, r, :] = t[0:COUT, :]


def _transpose_out(out_scat):
    return pl.pallas_call(
        _tr_body,
        grid=(OH // 8,),
        in_specs=[pl.BlockSpec((8 * OW, CP), lambda i: (i, 0))],
        out_specs=pl.BlockSpec((COUT, 8, OW), lambda i: (0, i, 0)),
        out_shape=jax.ShapeDtypeStruct((COUT, OH, OW), jnp.float32),
    )(out_scat)


_MESH = plsc.VectorSubcoreMesh(core_axis_name="c", subcore_axis_name="s")


@functools.partial(
    pl.kernel,
    out_type=jax.ShapeDtypeStruct((NPIX, CP), jnp.float32),
    mesh=_MESH,
    scratch_types=[
        pltpu.VMEM_SHARED((ACC_ROWS, CP), jnp.float32),
    ],
    compiler_params=pltpu.CompilerParams(needs_layout_passes=False),
)
def _sc_scatter(contrib_hbm, flat_hbm, bias_hbm, out_hbm, accum):
    pl.run_scoped(
        functools.partial(_sc_body, contrib_hbm, flat_hbm, bias_hbm,
                          out_hbm, accum),
        pltpu.VMEM((2, WROWS, 128), jnp.int32),
        pltpu.VMEM((MBUF,), jnp.int32),
        pltpu.VMEM((MBUF,), jnp.int32),
        pltpu.VMEM((2, GB), jnp.int32),
        pltpu.VMEM((2, GB), jnp.int32),
        pltpu.VMEM((2, GB, CP), jnp.float32),
        pltpu.VMEM((32, CP), jnp.float32),
        pltpu.SemaphoreType.DMA,
        pltpu.SemaphoreType.DMA,
    )


def _sc_body(contrib_hbm, flat_hbm, bias_hbm, out_hbm, accum,
             flat_win, dst_buf, src_buf, dst_batch, src_batch,
             staging, bias_v, sem, sem_f):
    c = lax.axis_index("c")
    s = lax.axis_index("s")
    iota16 = lax.iota(jnp.int32, 16)
    pltpu.sync_copy(bias_hbm, bias_v)

    def drain_prev(nfired):
        prev = lax.rem(nfired + 1, 2)

        @pl.when(nfired >= 1)
        def _():
            pltpu.make_async_copy(contrib_hbm.at[src_batch.at[prev]],
                                  staging.at[prev], sem).wait()
            pltpu.sync_copy(staging.at[prev], accum.at[dst_batch.at[prev]],
                            add=True)

    def flush_one(j, nfired):
        drain_prev(nfired)
        par = lax.rem(nfired, 2)
        base = j * GB
        for u in range(GB // 16):
            dst_batch[par, pl.ds(u * 16, 16)] = dst_buf[pl.ds(base + u * 16, 16)]
            src_batch[par, pl.ds(u * 16, 16)] = src_buf[pl.ds(base + u * 16, 16)]
        pltpu.async_copy(contrib_hbm.at[src_batch.at[par]], staging.at[par], sem)
        return nfired + 1

    def slab_body(t, carry):
        slab_id = 2 * t + c

        @pl.when(slab_id < NSLAB)
        def _():
            lo = slab_id * SLAB
            is_tail = slab_id == NSLAB - 1
            not_tail = slab_id != NSLAB - 1

            def init_fire(p, carry):
                pltpu.async_copy(
                    bias_v, accum.at[pl.ds(s * STRIPE + p * 32, 32)], sem_f)
                return carry

            def init_drain(p, carry):
                pltpu.make_async_copy(
                    bias_v, accum.at[pl.ds(s * STRIPE + p * 32, 32)],
                    sem_f).wait()
                return carry

            lax.fori_loop(0, STRIPE // 32, init_fire, 0)
            lax.fori_loop(0, STRIPE // 32, init_drain, 0)

            plsc.subcore_barrier()

            pltpu.async_copy(
                flat_hbm.at[pl.ds(s * CROWS, WROWS)], flat_win.at[0], sem_f)

            def win_body(w, carry2):
                cnt, nfired = carry2
                par = lax.rem(w, 2)
                pltpu.make_async_copy(
                    flat_hbm.at[pl.ds(s * CROWS + w * WROWS, WROWS)],
                    flat_win.at[par], sem_f).wait()

                @pl.when(w < NWIN - 1)
                def _():
                    pltpu.async_copy(
                        flat_hbm.at[pl.ds(s * CROWS + (w + 1) * WROWS, WROWS)],
                        flat_win.at[1 - par], sem_f)

                src_base = s * CHUNK + w * WSZ

                def scan_body(i, carry3):
                    cnt2, src_v = carry3
                    v = flat_win[par, i // 8, pl.ds((i % 8) * 16, 16)]
                    off = v - lo
                    b1 = lax.shift_right_logical(off, 31)
                    b2 = lax.shift_right_logical((SLAB - 1) - off, 31)
                    mi = (1 - b1) - b2
                    pref = plsc.cumsum(mi)
                    real_pos = cnt2 + pref - 1
                    pos = real_pos * mi + (WSZ + GB + iota16) * (1 - mi)
                    plsc.store_scatter(dst_buf, [pos], off)
                    plsc.store_scatter(src_buf, [pos], src_v)
                    return cnt2 + pref[15], src_v + 16

                cnt, _ = lax.fori_loop(0, WSZ // 16, scan_body,
                                       (cnt, src_base + iota16))

                nfl = cnt // GB
                nfired = lax.fori_loop(0, nfl, flush_one, nfired)
                rem_base = nfl * GB
                for u in range(GB // 16):
                    flat_win[par, 0, pl.ds(u * 16, 16)] = dst_buf[pl.ds(rem_base + u * 16, 16)]
                    flat_win[par, 1, pl.ds(u * 16, 16)] = src_buf[pl.ds(rem_base + u * 16, 16)]
                for u in range(GB // 16):
                    dst_buf[pl.ds(u * 16, 16)] = flat_win[par, 0, pl.ds(u * 16, 16)]
                    src_buf[pl.ds(u * 16, 16)] = flat_win[par, 1, pl.ds(u * 16, 16)]
                return cnt - nfl * GB, nfired

            cnt, nfired = lax.fori_loop(0, NWIN, win_body,
                                        (jnp.int32(0), jnp.int32(0)))

            pad_dst = jnp.full((16,), SLAB, jnp.int32)
            pad_src = jnp.zeros((16,), jnp.int32)
            for u in range(GB // 16):
                dst_buf[pl.ds(cnt + u * 16, 16)] = pad_dst
                src_buf[pl.ds(cnt + u * 16, 16)] = pad_src
            nfired = lax.fori_loop(0, (cnt + (GB - 1)) // GB, flush_one, nfired)
            drain_prev(nfired)

            plsc.subcore_barrier()

            def wout_fire(p, carry):
                pltpu.async_copy(
                    accum.at[pl.ds(s * STRIPE + p * 64, 64)],
                    out_hbm.at[pl.ds(lo + s * STRIPE + p * 64, 64)], sem_f)
                return carry

            def wout_drain(p, carry):
                pltpu.make_async_copy(
                    accum.at[pl.ds(s * STRIPE + p * 64, 64)],
                    out_hbm.at[pl.ds(lo + s * STRIPE + p * 64, 64)],
                    sem_f).wait()
                return carry

            lax.fori_loop(0, STRIPE // 64, wout_fire, 0)
            lax.fori_loop(0, STRIPE // 64, wout_drain, 0)

            plsc.subcore_barrier()

        return carry

    lax.fori_loop(0, SLABS_PER_CORE, slab_body, 0)


def kernel(x, weight, bias, sample_map, oh, ow):
    wt = jnp.transpose(weight, (0, 2, 1))
    w2p = jnp.pad(wt, ((0, 0), (0, 0), (0, CP - COUT))).reshape(CIN, KT * CP)
    smk = sample_map.reshape(P, KT, 2).transpose(1, 0, 2)

    contrib, flat2d = _tc_matmul(x, w2p, smk)
    biasp = jnp.pad(bias, (0, CP - COUT))
    bias_tile = jnp.tile(biasp[None, :], (32, 1))

    out_scat = _sc_scatter(contrib, flat2d, bias_tile)
    out = _transpose_out(out_scat)
    return out[None]


# speedup vs baseline: 1.3604x; 1.0055x over previous
import functools

import jax
import jax.numpy as jnp
from jax import lax
from jax.experimental import pallas as pl
from jax.experimental.pallas import tpu as pltpu
from jax.experimental.pallas import tpu_sc as plsc

CIN = 96
COUT = 96
KT = 4
H = W = 384
P = H * W
N = KT * P
OH, OW = 2 * H, 2 * W
NPIX = OH * OW
CP = 128

RPX = 8 * W
GI = P // RPX
FROWS = N // 128

NTILES = 16
CHUNK = N // NTILES
CROWS = CHUNK // 128
SLAB = 12288
NSLAB = (NPIX + SLAB - 1) // SLAB
TAIL = NPIX - (NSLAB - 1) * SLAB
ACC_ROWS = SLAB + 16
STRIPE = SLAB // NTILES
TAIL_STRIPE = TAIL // NTILES
GB = 64
SLABS_PER_CORE = (NSLAB + 1) // 2
WSZ = 2048
WROWS = WSZ // 128
NWIN = CHUNK // WSZ
MBUF = WSZ + 2 * GB


def _tc_body(x_ref, w_ref, sm_ref, contrib_ref, flat_ref):
    wb = w_ref[...]
    for r in range(8):
        xr = x_ref[0, :, r, :]
        res = lax.dot_general(xr, wb, (((0,), (0,)), ((), ())),
                              preferred_element_type=jnp.float32)
        contrib_ref[pl.ds(r * W, W), :] = res
    sm = sm_ref[0]
    fl = sm[:, 1] * OW + sm[:, 0]
    flat_ref[...] = fl.reshape(RPX // 128, 128)


def _tc_matmul(x, w2p, smk):
    return pl.pallas_call(
        _tc_body,
        grid=(GI, KT),
        in_specs=[
            pl.BlockSpec((1, CIN, 8, W), lambda i, k: (0, 0, i, 0)),
            pl.BlockSpec((CIN, CP), lambda i, k: (0, k)),
            pl.BlockSpec((1, RPX, 2), lambda i, k: (k, i, 0)),
        ],
        out_specs=[
            pl.BlockSpec((RPX, CP), lambda i, k: (k * GI + i, 0)),
            pl.BlockSpec((RPX // 128, 128), lambda i, k: (k * GI + i, 0)),
        ],
        out_shape=[
            jax.ShapeDtypeStruct((N, CP), jnp.float32),
            jax.ShapeDtypeStruct((FROWS, 128), jnp.int32),
        ],
    )(x, w2p, smk)


def _tr_body(in_ref, out_ref):
    for r in range(8):
        blk = in_ref[pl.ds(r * OW, OW), :]
        t = jnp.transpose(blk, (1, 0))
        out_ref[:, r, :] = t[0:COUT, :]


def _transpose_out(out_scat):
    return pl.pallas_call(
        _tr_body,
        grid=(OH // 8,),
        in_specs=[pl.BlockSpec((8 * OW, CP), lambda i: (i, 0))],
        out_specs=pl.BlockSpec((COUT, 8, OW), lambda i: (0, i, 0)),
        out_shape=jax.ShapeDtypeStruct((COUT, OH, OW), jnp.float32),
    )(out_scat)


_MESH = plsc.VectorSubcoreMesh(core_axis_name="c", subcore_axis_name="s")


@functools.partial(
    pl.kernel,
    out_type=jax.ShapeDtypeStruct((NPIX, CP), jnp.float32),
    mesh=_MESH,
    scratch_types=[
        pltpu.VMEM_SHARED((ACC_ROWS, CP), jnp.float32),
    ],
    compiler_params=pltpu.CompilerParams(needs_layout_passes=False),
)
def _sc_scatter(contrib_hbm, flat_hbm, bias_hbm, out_hbm, accum):
    pl.run_scoped(
        functools.partial(_sc_body, contrib_hbm, flat_hbm, bias_hbm,
                          out_hbm, accum),
        pltpu.VMEM((2, WROWS, 128), jnp.int32),
        pltpu.VMEM((MBUF,), jnp.int32),
        pltpu.VMEM((MBUF,), jnp.int32),
        pltpu.VMEM((2, GB), jnp.int32),
        pltpu.VMEM((2, GB), jnp.int32),
        pltpu.VMEM((2, GB, CP), jnp.float32),
        pltpu.VMEM((32, CP), jnp.float32),
        pltpu.SemaphoreType.DMA,
        pltpu.SemaphoreType.DMA,
    )


def _sc_body(contrib_hbm, flat_hbm, bias_hbm, out_hbm, accum,
             flat_win, dst_buf, src_buf, dst_batch, src_batch,
             staging, bias_v, sem, sem_f):
    c = lax.axis_index("c")
    s = lax.axis_index("s")
    iota16 = lax.iota(jnp.int32, 16)
    pltpu.sync_copy(bias_hbm, bias_v)

    def drain_prev(nfired):
        prev = lax.rem(nfired + 1, 2)

        @pl.when(nfired >= 1)
        def _():
            pltpu.make_async_copy(contrib_hbm.at[src_batch.at[prev]],
                                  staging.at[prev], sem).wait()
            pltpu.sync_copy(staging.at[prev], accum.at[dst_batch.at[prev]],
                            add=True)

    def flush_one(j, nfired):
        drain_prev(nfired)
        par = lax.rem(nfired, 2)
        base = j * GB
        for u in range(GB // 16):
            dst_batch[par, pl.ds(u * 16, 16)] = dst_buf[pl.ds(base + u * 16, 16)]
            src_batch[par, pl.ds(u * 16, 16)] = src_buf[pl.ds(base + u * 16, 16)]
        pltpu.async_copy(contrib_hbm.at[src_batch.at[par]], staging.at[par], sem)
        return nfired + 1

    def slab_body(t, carry):
        slab_id = 2 * t + c

        @pl.when(slab_id < NSLAB)
        def _():
            lo = slab_id * SLAB
            is_tail = slab_id == NSLAB - 1
            not_tail = slab_id != NSLAB - 1

            def init_fire(p, carry):
                pltpu.async_copy(
                    bias_v, accum.at[pl.ds(s * STRIPE + p * 32, 32)], sem_f)
                return carry

            def init_drain(p, carry):
                pltpu.make_async_copy(
                    bias_v, accum.at[pl.ds(s * STRIPE + p * 32, 32)],
                    sem_f).wait()
                return carry

            lax.fori_loop(0, STRIPE // 32, init_fire, 0)
            lax.fori_loop(0, STRIPE // 32, init_drain, 0)

            plsc.subcore_barrier()

            pltpu.async_copy(
                flat_hbm.at[pl.ds(s * CROWS, WROWS)], flat_win.at[0], sem_f)

            def win_body(w, carry2):
                cnt, nfired = carry2
                par = lax.rem(w, 2)
                pltpu.make_async_copy(
                    flat_hbm.at[pl.ds(s * CROWS + w * WROWS, WROWS)],
                    flat_win.at[par], sem_f).wait()

                @pl.when(w < NWIN - 1)
                def _():
                    pltpu.async_copy(
                        flat_hbm.at[pl.ds(s * CROWS + (w + 1) * WROWS, WROWS)],
                        flat_win.at[1 - par], sem_f)

                src_base = s * CHUNK + w * WSZ

                def scan_body(i, carry3):
                    cnt2, src_v = carry3
                    vA = flat_win[par, i // 4, pl.ds((i % 4) * 32, 16)]
                    vB = flat_win[par, i // 4, pl.ds((i % 4) * 32 + 16, 16)]
                    offA = vA - lo
                    offB = vB - lo
                    mA = plsc.bitcast(offA, jnp.uint32) < jnp.uint32(SLAB)
                    mB = plsc.bitcast(offB, jnp.uint32) < jnp.uint32(SLAB)
                    prefA = plsc.cumsum(mA.astype(jnp.int32))
                    prefB = plsc.cumsum(mB.astype(jnp.int32))
                    nA = prefA[15]
                    posA = cnt2 + prefA - 1
                    posB = (cnt2 + nA) + prefB - 1
                    plsc.store_scatter(dst_buf, [posA], offA, mask=mA)
                    plsc.store_scatter(src_buf, [posA], src_v, mask=mA)
                    plsc.store_scatter(dst_buf, [posB], offB, mask=mB)
                    plsc.store_scatter(src_buf, [posB], src_v + 16, mask=mB)
                    return (cnt2 + nA) + prefB[15], src_v + 32

                cnt, _ = lax.fori_loop(0, WSZ // 32, scan_body,
                                       (cnt, src_base + iota16))

                nfl = cnt // GB
                nfired = lax.fori_loop(0, nfl, flush_one, nfired)
                rem_base = nfl * GB
                for u in range(GB // 16):
                    flat_win[par, 0, pl.ds(u * 16, 16)] = dst_buf[pl.ds(rem_base + u * 16, 16)]
                    flat_win[par, 1, pl.ds(u * 16, 16)] = src_buf[pl.ds(rem_base + u * 16, 16)]
                for u in range(GB // 16):
                    dst_buf[pl.ds(u * 16, 16)] = flat_win[par, 0, pl.ds(u * 16, 16)]
                    src_buf[pl.ds(u * 16, 16)] = flat_win[par, 1, pl.ds(u * 16, 16)]
                return cnt - nfl * GB, nfired

            cnt, nfired = lax.fori_loop(0, NWIN, win_body,
                                        (jnp.int32(0), jnp.int32(0)))

            pad_dst = jnp.full((16,), SLAB, jnp.int32)
            pad_src = jnp.zeros((16,), jnp.int32)
            for u in range(GB // 16):
                dst_buf[pl.ds(cnt + u * 16, 16)] = pad_dst
                src_buf[pl.ds(cnt + u * 16, 16)] = pad_src
            nfired = lax.fori_loop(0, (cnt + (GB - 1)) // GB, flush_one, nfired)
            drain_prev(nfired)

            plsc.subcore_barrier()

            def wout_fire(p, carry):
                pltpu.async_copy(
                    accum.at[pl.ds(s * STRIPE + p * 64, 64)],
                    out_hbm.at[pl.ds(lo + s * STRIPE + p * 64, 64)], sem_f)
                return carry

            def wout_drain(p, carry):
                pltpu.make_async_copy(
                    accum.at[pl.ds(s * STRIPE + p * 64, 64)],
                    out_hbm.at[pl.ds(lo + s * STRIPE + p * 64, 64)],
                    sem_f).wait()
                return carry

            lax.fori_loop(0, STRIPE // 64, wout_fire, 0)
            lax.fori_loop(0, STRIPE // 64, wout_drain, 0)

            plsc.subcore_barrier()

        return carry

    lax.fori_loop(0, SLABS_PER_CORE, slab_body, 0)


def kernel(x, weight, bias, sample_map, oh, ow):
    wt = jnp.transpose(weight, (0, 2, 1))
    w2p = jnp.pad(wt, ((0, 0), (0, 0), (0, CP - COUT))).reshape(CIN, KT * CP)
    smk = sample_map.reshape(P, KT, 2).transpose(1, 0, 2)

    contrib, flat2d = _tc_matmul(x, w2p, smk)
    biasp = jnp.pad(bias, (0, CP - COUT))
    bias_tile = jnp.tile(biasp[None, :], (32, 1))

    out_scat = _sc_scatter(contrib, flat2d, bias_tile)
    out = _transpose_out(out_scat)
    return out[None]
